# baseline (speedup 1.0000x reference)
import functools

import numpy as np
import jax
import jax.numpy as jnp
from jax import lax
from jax.experimental import pallas as pl
from jax.experimental.pallas import tpu as pltpu
from jax.experimental.pallas import tpu_sc as plsc

_OMEGA_SCALE = 10000


def _pos_table(seqlen, dims):
    positions = np.arange(seqlen)[:, np.newaxis]
    omega = 1 / np.power(_OMEGA_SCALE, 2 * (np.arange(dims, dtype=int) // 2) / np.float64(dims))
    emb = positions * omega
    emb[:, 0::2] = np.sin(emb[:, 0::2])
    emb[:, 1::2] = np.cos(emb[:, 1::2])
    return emb.astype(np.float32)


@functools.lru_cache(maxsize=None)
def _make_repack(V, D):
    info = plsc.get_sparse_core_info()
    NC, NS, L = info.num_cores, info.num_subcores, info.num_lanes
    NW = NC * NS
    VN = D // L
    R = 128
    GROUPS = V // 16
    GPC = R // 16

    mesh = plsc.VectorSubcoreMesh(core_axis_name="c", subcore_axis_name="s")

    @functools.partial(
        pl.kernel, mesh=mesh,
        out_type=jax.ShapeDtypeStruct((V // 2, 2 * D), jnp.float32),
        scratch_types=[
            pltpu.VMEM((R, D), jnp.float32),
            pltpu.VMEM((R // 2, 2 * D), jnp.float32),
        ],
    )
    def k(t_hbm, out_hbm, in_v, out_v):
        wid = lax.axis_index("s") * NC + lax.axis_index("c")
        gq, gr = GROUPS // NW, GROUPS % NW
        ng = gq + jnp.where(wid < gr, 1, 0)
        g0 = wid * gq + jnp.minimum(wid, gr)

        def copy_rows(row0, nrows):
            pltpu.sync_copy(t_hbm.at[pl.ds(pl.multiple_of(row0, 16), nrows)],
                            in_v.at[pl.ds(0, nrows)])
            def pair_body(j, acc):
                for v in range(VN):
                    out_v[j, pl.ds(v * L, L)] = in_v[2 * j, pl.ds(v * L, L)]
                    out_v[j, pl.ds(D + v * L, L)] = in_v[2 * j + 1, pl.ds(v * L, L)]
                return acc
            lax.fori_loop(0, nrows // 2, pair_body, 0)
            pltpu.sync_copy(out_v.at[pl.ds(0, nrows // 2)],
                            out_hbm.at[pl.ds(pl.multiple_of(row0 // 2, 8), nrows // 2)])

        def chunk_body(c, carry):
            copy_rows((g0 + c * GPC) * 16, R)
            return carry

        n_full = ng // GPC
        lax.fori_loop(0, n_full, chunk_body, 0)

        def tail_body(t, carry):
            copy_rows((g0 + n_full * GPC + t) * 16, 16)
            return carry

        lax.fori_loop(0, ng - n_full * GPC, tail_body, 0)

    return k


@functools.lru_cache(maxsize=None)
def _make_embed(V, D, N, S):
    info = plsc.get_sparse_core_info()
    NC, NS, L = info.num_cores, info.num_subcores, info.num_lanes
    NW = NC * NS
    n_per_w = N // NW
    W = 80
    G = 4
    C = G * W
    n_chunks = n_per_w // C
    VN = D // L
    D2 = 2 * D
    assert N % (NW * C) == 0 and D % L == 0 and W % L == 0

    mesh = plsc.VectorSubcoreMesh(core_axis_name="c", subcore_axis_name="s")

    @functools.partial(
        pl.kernel, mesh=mesh,
        compiler_params=pltpu.CompilerParams(use_tc_tiling_on_sc=False),
        out_type=jax.ShapeDtypeStruct((N, D), jnp.float32),
        scratch_types=[
            pltpu.VMEM((G, W), jnp.int32),
            pltpu.VMEM((G, W), jnp.int32),
            pltpu.VMEM((C, D2), jnp.float32),
            pltpu.VMEM((C, D), jnp.float32),
            pltpu.VMEM((S, D), jnp.float32),
            pltpu.SemaphoreType.DMA,
        ],
    )
    def k(table_hbm, idx_hbm, pos_hbm, out_hbm,
          idx_v, pair_v, rows_v, out_v, pos_v, sem):
        wid = lax.axis_index("s") * NC + lax.axis_index("c")
        blk0 = wid * n_chunks
        pltpu.sync_copy(pos_hbm, pos_v)

        def chunk_body(c, carry):
            blk = blk0 + c
            cbase = blk * C
            pltpu.sync_copy(idx_hbm.at[blk], idx_v)
            for j in range(G):
                for u in range(W // L):
                    sl = pl.ds(u * L, L)
                    pair_v[j, sl] = lax.shift_right_logical(idx_v[j, sl], 1)
            copies = [
                pltpu.async_copy(table_hbm.at[pair_v.at[j]],
                                 rows_v.at[pl.ds(j * W, W)], sem)
                for j in range(G)
            ]
            for cp in copies:
                cp.wait()

            def blk16_body(t, s0):
                base = t * L
                par = idx_v[base // W, pl.ds(base % W, L)] & 1
                for l in range(L):
                    i = base + l
                    half = par[l] * D
                    s = lax.select(s0 + l >= S, s0 + l - S, s0 + l)
                    for v in range(VN):
                        out_v[i, pl.ds(v * L, L)] = (
                            rows_v[i, pl.ds(half + v * L, L)]
                            + pos_v[s, pl.ds(v * L, L)])
                s0 = s0 + L
                return lax.select(s0 >= S, s0 - S, s0)

            lax.fori_loop(0, C // L, blk16_body, lax.rem(cbase, S))
            pltpu.sync_copy(out_v, out_hbm.at[pl.ds(cbase, C)])
            return carry

        lax.fori_loop(0, n_chunks, chunk_body, 0)

    return k


def kernel(inputs, table):
    B, S = inputs.shape
    V, D = table.shape
    N = B * S
    idx3d = inputs.reshape(N // 320, 4, 80).astype(jnp.int32)
    pos = jnp.asarray(_pos_table(S, D))
    t128 = _make_repack(V, D)(table)
    out = _make_embed(V, D, N, S)(t128, idx3d, pos)
    return out.reshape(B, S, D)


# speedup vs baseline: 1.7544x; 1.7544x over previous
import functools

import numpy as np
import jax
import jax.numpy as jnp
from jax import lax
from jax.experimental import pallas as pl
from jax.experimental.pallas import tpu as pltpu
from jax.experimental.pallas import tpu_sc as plsc

_OMEGA_SCALE = 10000


def _pos_table(seqlen, dims):
    positions = np.arange(seqlen)[:, np.newaxis]
    omega = 1 / np.power(_OMEGA_SCALE, 2 * (np.arange(dims, dtype=int) // 2) / np.float64(dims))
    emb = positions * omega
    emb[:, 0::2] = np.sin(emb[:, 0::2])
    emb[:, 1::2] = np.cos(emb[:, 1::2])
    return emb.astype(np.float32)


@functools.lru_cache(maxsize=None)
def _make_embed(V, D, N, S):
    info = plsc.get_sparse_core_info()
    NC, NS, L = info.num_cores, info.num_subcores, info.num_lanes
    NW = NC * NS
    n_per_w = N // NW
    W = 80
    G = 8
    C = G * W
    QF = C // S
    QT = C % S
    n_chunks = n_per_w // C
    VN = D // L
    assert N % (NW * C) == 0 and D % L == 0 and W % L == 0

    mesh = plsc.VectorSubcoreMesh(core_axis_name="c", subcore_axis_name="s")

    @functools.partial(
        pl.kernel, mesh=mesh,
        compiler_params=pltpu.CompilerParams(use_tc_tiling_on_sc=False),
        out_type=jax.ShapeDtypeStruct((N, D), jnp.float32),
        scratch_types=[
            pltpu.VMEM((2, G // 2, W), jnp.int32),
            pltpu.VMEM((C, D), jnp.float32),
            pltpu.VMEM((S, D), jnp.float32),
            pltpu.SemaphoreType.DMA,
        ],
    )
    def k(table_hbm, idx_hbm, pos_hbm, out_hbm, idx_v, rows_v, pos_v, sem):
        wid = lax.axis_index("s") * NC + lax.axis_index("c")
        blk0 = wid * (n_chunks * 2)
        pltpu.sync_copy(pos_hbm, pos_v)

        def add_row(r, s):
            for v in range(VN):
                plsc.addupdate(rows_v.at[r, pl.ds(v * L, L)],
                               pos_v[s, pl.ds(v * L, L)])

        def chunk_body(c, s0):
            cbase = blk0 * (C // 2) + c * C
            pltpu.sync_copy(idx_hbm.at[pl.ds(blk0 + c * 2, 2)], idx_v)
            copies = [
                pltpu.async_copy(table_hbm.at[idx_v.at[j // (G // 2), j % (G // 2)]],
                                 rows_v.at[pl.ds(j * W, W)], sem)
                for j in range(G)
            ]
            for cp in copies:
                cp.wait()

            def s_body(s, acc):
                r0 = lax.select(s >= s0, s - s0, s - s0 + S)
                for q in range(QF):
                    add_row(r0 + q * S, s)

                @pl.when(r0 < QT)
                def _():
                    add_row(r0 + QF * S, s)
                return acc

            lax.fori_loop(0, S, s_body, 0)
            pltpu.sync_copy(rows_v, out_hbm.at[pl.ds(cbase, C)])
            s0 = s0 + QT
            return lax.select(s0 >= S, s0 - S, s0)

        lax.fori_loop(0, n_chunks, chunk_body, lax.rem(blk0 * (C // 2), S))

    return k


def kernel(inputs, table):
    B, S = inputs.shape
    V, D = table.shape
    N = B * S
    idx3d = inputs.reshape(N // 320, 4, 80).astype(jnp.int32)
    pos = jnp.asarray(_pos_table(S, D))
    out = _make_embed(V, D, N, S)(table, idx3d, pos)
    return out.reshape(B, S, D)


# speedup vs baseline: 1.9662x; 1.1207x over previous
import functools

import numpy as np
import jax
import jax.numpy as jnp
from jax import lax
from jax.experimental import pallas as pl
from jax.experimental.pallas import tpu as pltpu
from jax.experimental.pallas import tpu_sc as plsc

_OMEGA_SCALE = 10000


def _pos_table(seqlen, dims):
    positions = np.arange(seqlen)[:, np.newaxis]
    omega = 1 / np.power(_OMEGA_SCALE, 2 * (np.arange(dims, dtype=int) // 2) / np.float64(dims))
    emb = positions * omega
    emb[:, 0::2] = np.sin(emb[:, 0::2])
    emb[:, 1::2] = np.cos(emb[:, 1::2])
    return emb.astype(np.float32)


@functools.lru_cache(maxsize=None)
def _make_embed(V, D, N, S):
    info = plsc.get_sparse_core_info()
    NC, NS, L = info.num_cores, info.num_subcores, info.num_lanes
    NW = NC * NS
    n_per_w = N // NW
    W = 100
    C = 4 * S
    G = C // W
    n_chunks = n_per_w // C
    VN = D // L
    assert N % NW == 0 and n_per_w % C == 0 and C % W == 0 and C % S == 0
    assert N % W == 0 and D % L == 0 and G % 8 == 0

    mesh = plsc.VectorSubcoreMesh(core_axis_name="c", subcore_axis_name="s")

    @functools.partial(
        pl.kernel, mesh=mesh,
        compiler_params=pltpu.CompilerParams(use_tc_tiling_on_sc=False),
        out_type=jax.ShapeDtypeStruct((N, D), jnp.float32),
        scratch_types=[
            pltpu.VMEM((G, W), jnp.int32),
            pltpu.VMEM((C, D), jnp.float32),
            pltpu.VMEM((S, D), jnp.float32),
            pltpu.SemaphoreType.DMA,
        ],
    )
    def k(table_hbm, idx_hbm, pos_hbm, out_hbm, idx_v, rows_v, pos_v, sem):
        wid = lax.axis_index("s") * NC + lax.axis_index("c")
        base = wid * n_per_w
        pltpu.sync_copy(pos_hbm, pos_v)

        def chunk_body(c, carry):
            cbase = pl.multiple_of(base + c * C, C)
            pltpu.sync_copy(idx_hbm.at[pl.ds(pl.multiple_of(cbase // W, 8), G)], idx_v)
            copies = [
                pltpu.async_copy(table_hbm.at[idx_v.at[j]],
                                 rows_v.at[pl.ds(j * W, W)], sem)
                for j in range(G)
            ]
            for cp in copies:
                cp.wait()

            def add_body(s, acc):
                for v in range(VN):
                    p = pos_v[s, pl.ds(v * L, L)]
                    for q in range(C // S):
                        plsc.addupdate(rows_v.at[q * S + s, pl.ds(v * L, L)], p)
                return acc

            lax.fori_loop(0, S, add_body, 0)
            pltpu.sync_copy(rows_v, out_hbm.at[pl.ds(cbase, C)])
            return carry

        lax.fori_loop(0, n_chunks, chunk_body, 0)

    return k


def kernel(inputs, table):
    B, S = inputs.shape
    V, D = table.shape
    N = B * S
    flat = inputs.reshape(N).astype(jnp.int32)
    idx2d = flat.reshape(N // 100, 100)
    pos = jnp.asarray(_pos_table(S, D))
    out = _make_embed(V, D, N, S)(table, idx2d, pos)
    return out.reshape(B, S, D)
